# baseline (speedup 1.0000x reference)
import functools

import jax
import jax.numpy as jnp
from jax import lax
from jax.experimental import pallas as pl
from jax.experimental.pallas import tpu as pltpu
from jax.experimental.pallas import tpu_sc as plsc

_BETA = 5.5
_LOGIT_SCALE = 100.0
_C, _M, _D = 1000, 50, 1024
_NW = 32
_CPW = 8
_S = _NW * _CPW
_NCH = _D // 16
_TCB = 8


def _bsum(v):
    return jnp.broadcast_to(jnp.sum(v), (16,))


def _rsqrt(x):
    i = lax.bitcast_convert_type(x, jnp.int32)
    y = lax.bitcast_convert_type(jnp.int32(0x5F3759DF) - (i >> 1), jnp.float32)
    for _ in range(3):
        y = y * (1.5 - 0.5 * x * y * y)
    return y


def _sc_body(q_hbm, mem_hbm, fx_hbm, bk_hbm, bv_hbm, ffn_hbm, img_hbm,
             out_hbm, rows_v, bkv, bvv, fnv, fxv, adap_v, qv_v, iv_v, lg_v,
             sem_a):
    wid = lax.axis_index("s") * 2 + lax.axis_index("c")
    z = jnp.zeros((16,), jnp.float32)
    base = wid * _CPW

    pltpu.sync_copy(q_hbm, qv_v)
    pltpu.sync_copy(img_hbm, iv_v)
    pltpu.sync_copy(bk_hbm.at[pl.ds(base, _CPW)], bkv)
    pltpu.sync_copy(bv_hbm.at[pl.ds(base, _CPW)], bvv)
    pltpu.sync_copy(ffn_hbm.at[pl.ds(base, _CPW)], fnv)
    pltpu.sync_copy(fx_hbm.at[pl.ds(base, _CPW)], fxv)

    def _row_weight(accs, bkbk, bvbv, qbk):
        aq, ab, av, ar, asm = accs
        rr = _bsum(ar)
        kk = rr + 2.0 * _bsum(ab) + bkbk
        vv = rr + 2.0 * _bsum(av) + bvbv
        s = _bsum(aq) + qbk
        sim = jnp.exp(_BETA * (s * _rsqrt(kk) - 1.0))
        return jnp.where(_bsum(asm) == 0.0, 0.0, sim * _rsqrt(vv))

    def _compute(k_idx):
        def _cc(ch, acc):
            a1, a2, a3 = acc
            sl = pl.ds(ch * 16, 16)
            qc = qv_v[0, sl]
            bkc = bkv[k_idx, sl]
            bvc = bvv[k_idx, sl]
            return (a1 + bkc * bkc, a2 + bvc * bvc, a3 + qc * bkc)
        bkbk, bvbv, qbk = lax.fori_loop(0, _NCH, _cc, (z, z, z), unroll=2)
        bkbk = _bsum(bkbk)
        bvbv = _bsum(bvbv)
        qbk = _bsum(qbk)

        def _p1_block(loads):
            nr = len(loads)

            def _p1(ch, acc):
                sl = pl.ds(ch * 16, 16)
                qc = qv_v[0, sl]
                bkc = bkv[k_idx, sl]
                bvc = bvv[k_idx, sl]
                out = []
                for j in range(nr):
                    rv = loads[j](sl)
                    aq, ab, av, ar, asm = acc[j]
                    out.append((aq + rv * qc, ab + rv * bkc, av + rv * bvc,
                                ar + rv * rv, asm + rv))
                return tuple(out)
            res = lax.fori_loop(0, _NCH, _p1, tuple((z, z, z, z, z)
                                                    for _ in range(nr)),
                                unroll=2)
            return [_row_weight(res[j], bkbk, bvbv, qbk) for j in range(nr)]

        lloads = [lambda sl: rows_v[48, sl], lambda sl: rows_v[49, sl],
                  lambda sl: fxv[k_idx, 0, sl]]
        lw = _p1_block(lloads)
        wsum = lw[0] + lw[1] + lw[2]

        def _p2l(ch, carry):
            sl = pl.ds(ch * 16, 16)
            adap_v[sl] = (rows_v[48, sl] * lw[0] +
                          rows_v[49, sl] * lw[1] + fxv[k_idx, 0, sl] * lw[2])
            return carry
        lax.fori_loop(0, _NCH, _p2l, 0, unroll=2)

        for r0 in range(0, 48, 6):
            ws = _p1_block([(lambda sl, r=r0 + j: rows_v[r, sl])
                            for j in range(6)])
            for w in ws:
                wsum = wsum + w

            def _p2(ch, carry, r0=r0, ws=ws):
                sl = pl.ds(ch * 16, 16)
                acc = adap_v[sl]
                for j in range(6):
                    acc = acc + rows_v[r0 + j, sl] * ws[j]
                adap_v[sl] = acc
                return carry
            lax.fori_loop(0, _NCH, _p2, 0, unroll=2)

        def _p3a(ch, acc):
            sl = pl.ds(ch * 16, 16)
            x = adap_v[sl] + wsum * bvv[k_idx, sl]
            return acc + x * x
        aa = _bsum(lax.fori_loop(0, _NCH, _p3a, z, unroll=2))
        r1 = _rsqrt(aa)

        def _p3b(ch, acc):
            a2, ai = acc
            sl = pl.ds(ch * 16, 16)
            x = (adap_v[sl] + wsum * bvv[k_idx, sl]) * r1 + fnv[k_idx, sl]
            return (a2 + x * x, ai + x * iv_v[0, sl])
        aa2, ai = lax.fori_loop(0, _NCH, _p3b, (z, z), unroll=2)
        lg_v[k_idx, :] = _LOGIT_SCALE * _bsum(ai) * _rsqrt(_bsum(aa2))

    def _class_body(k_idx, carry):
        pltpu.async_copy(mem_hbm.at[base + k_idx], rows_v, sem_a).wait()
        _compute(k_idx)
        return carry
    lax.fori_loop(0, _CPW, _class_body, 0)
    pltpu.sync_copy(lg_v, out_hbm.at[pl.ds(base, _CPW)])


def _tc_body(q_ref, img_ref, mem_ref, fx_ref, bk_ref, bv_ref, fn_ref, o_ref):
    mem = mem_ref[...]
    fx = fx_ref[...]
    bk = bk_ref[...]
    bv = bv_ref[...]
    fn = fn_ref[...]

    def _b3(x, shape, dims):
        return lax.broadcast_in_dim(x, shape, dims)

    q3 = _b3(q_ref[...], (_TCB, _M, _D), (0, 2))
    bk3 = _b3(bk, (_TCB, _M, _D), (0, 2))
    bv3 = _b3(bv, (_TCB, _M, _D), (0, 2))
    qf3 = _b3(q_ref[...], (_TCB, 1, _D), (0, 2))
    bkf3 = _b3(bk, (_TCB, 1, _D), (0, 2))
    bvf3 = _b3(bv, (_TCB, 1, _D), (0, 2))
    bkbk = jnp.sum(bk * bk, -1, keepdims=True)
    bvbv = jnp.sum(bv * bv, -1, keepdims=True)
    qbk = jnp.sum(bk * _b3(q_ref[...], (_TCB, _D), (0, 1)), -1, keepdims=True)

    def _w(r, qx, bkx, bvx):
        rr = jnp.sum(r * r, -1)
        kk = rr + 2.0 * jnp.sum(r * bkx, -1) + bkbk
        vv = rr + 2.0 * jnp.sum(r * bvx, -1) + bvbv
        s = jnp.sum(r * qx, -1) + qbk
        sim = jnp.exp(_BETA * (s * lax.rsqrt(kk) - 1.0))
        empty = jnp.sum(r, -1) == 0.0
        return jnp.where(empty, 0.0, sim * lax.rsqrt(vv))

    w = _w(mem, q3, bk3, bv3)
    w_f = _w(fx, qf3, bkf3, bvf3)
    a = jnp.sum(mem * _b3(w, (_TCB, _M, _D), (0, 1)), 1) + \
        jnp.sum(fx * _b3(w_f, (_TCB, 1, _D), (0, 1)), 1) + \
        (jnp.sum(w, -1, keepdims=True) + w_f) * bv
    a = a * lax.rsqrt(jnp.sum(a * a, -1, keepdims=True))
    a2 = a + fn
    img2 = _b3(img_ref[...], (_TCB, _D), (0, 1))
    o_ref[...] = _LOGIT_SCALE * jnp.sum(a2 * img2, -1, keepdims=True) * \
        lax.rsqrt(jnp.sum(a2 * a2, -1, keepdims=True))


def _q_body(img_ref, gb_ref, o_ref):
    s = jnp.sum(gb_ref[...], axis=0, keepdims=True) * (1.0 / _C) + img_ref[...]
    o_ref[...] = s * lax.rsqrt(jnp.sum(s * s))


def _softmax_body(x_ref, o_ref):
    x = x_ref[...]
    idx = lax.broadcasted_iota(jnp.int32, (8, 128), 0) * 128 + \
        lax.broadcasted_iota(jnp.int32, (8, 128), 1)
    x = jnp.where(idx < _C, x, -jnp.inf)
    e = jnp.where(idx < _C, jnp.exp(x - jnp.max(x)), 0.0)
    o_ref[...] = e / jnp.sum(e)


@jax.jit
def kernel(img_feat, image_feature_memory, fixed_global_feat_vanilla,
           global_bias, global_bias_key, global_bias_value, global_ffn_bias):
    q = pl.pallas_call(
        _q_body,
        out_shape=jax.ShapeDtypeStruct((1, _D), jnp.float32),
    )(img_feat, global_bias)

    sc = pl.kernel(
        _sc_body,
        mesh=plsc.VectorSubcoreMesh(core_axis_name="c", subcore_axis_name="s"),
        out_type=jax.ShapeDtypeStruct((_S, 16), jnp.float32),
        compiler_params=pltpu.CompilerParams(use_tc_tiling_on_sc=True,
                                             needs_layout_passes=False),
        scratch_types=[
            pltpu.VMEM((_M, _D), jnp.float32),
            pltpu.VMEM((_CPW, _D), jnp.float32),
            pltpu.VMEM((_CPW, _D), jnp.float32),
            pltpu.VMEM((_CPW, _D), jnp.float32),
            pltpu.VMEM((_CPW, 1, _D), jnp.float32),
            pltpu.VMEM((_D,), jnp.float32),
            pltpu.VMEM((1, _D), jnp.float32),
            pltpu.VMEM((1, _D), jnp.float32),
            pltpu.VMEM((_CPW, 16), jnp.float32),
            pltpu.SemaphoreType.DMA,
        ],
    )
    sc_lg16 = sc(q, image_feature_memory[:_S],
                 fixed_global_feat_vanilla[:_S], global_bias_key[:_S],
                 global_bias_value[:_S], global_ffn_bias[:_S], img_feat)

    ntc = (_C - _S) // _TCB
    tc_lg = pl.pallas_call(
        _tc_body,
        grid=(ntc,),
        in_specs=[
            pl.BlockSpec((1, _D), lambda i: (0, 0)),
            pl.BlockSpec((1, _D), lambda i: (0, 0)),
            pl.BlockSpec((_TCB, _M, _D), lambda i: (_S // _TCB + i, 0, 0)),
            pl.BlockSpec((_TCB, 1, _D), lambda i: (_S // _TCB + i, 0, 0)),
            pl.BlockSpec((_TCB, _D), lambda i: (_S // _TCB + i, 0)),
            pl.BlockSpec((_TCB, _D), lambda i: (_S // _TCB + i, 0)),
            pl.BlockSpec((_TCB, _D), lambda i: (_S // _TCB + i, 0)),
        ],
        out_specs=pl.BlockSpec((_TCB, 1), lambda i: (i, 0)),
        out_shape=jax.ShapeDtypeStruct((_C - _S, 1), jnp.float32),
    )(q, img_feat, image_feature_memory, fixed_global_feat_vanilla,
      global_bias_key, global_bias_value, global_ffn_bias)

    lg = jnp.concatenate([sc_lg16[:, 0], tc_lg[:, 0],
                          jnp.zeros(1024 - _C, jnp.float32)])
    probs = pl.pallas_call(
        _softmax_body,
        out_shape=jax.ShapeDtypeStruct((8, 128), jnp.float32),
    )(lg.reshape(8, 128))
    return probs.reshape(1024)[:_C][None, :]


# speedup vs baseline: 1.4315x; 1.4315x over previous
import functools

import jax
import jax.numpy as jnp
from jax import lax
from jax.experimental import pallas as pl
from jax.experimental.pallas import tpu as pltpu
from jax.experimental.pallas import tpu_sc as plsc

_BETA = 5.5
_LOGIT_SCALE = 100.0
_C, _M, _D = 1000, 50, 1024
_NW = 32
_CPW = 8
_S = _NW * _CPW
_NCH = _D // 16
_TCB = 8


def _bsum(v):
    return jnp.broadcast_to(jnp.sum(v), (16,))


def _rsqrt(x):
    i = lax.bitcast_convert_type(x, jnp.int32)
    y = lax.bitcast_convert_type(jnp.int32(0x5F3759DF) - (i >> 1), jnp.float32)
    for _ in range(3):
        y = y * (1.5 - 0.5 * x * y * y)
    return y


def _sc_body(q_hbm, mem_hbm, fx_hbm, bk_hbm, bv_hbm, ffn_hbm, img_hbm,
             out_hbm, rows_v, bkv, bvv, fnv, fxv, adap_v, qv_v, iv_v, lg_v,
             sem_a):
    wid = lax.axis_index("s") * 2 + lax.axis_index("c")
    z = jnp.zeros((16,), jnp.float32)
    base = wid * _CPW

    pltpu.sync_copy(q_hbm, qv_v)
    pltpu.sync_copy(img_hbm, iv_v)
    pltpu.sync_copy(bk_hbm.at[pl.ds(base, _CPW)], bkv)
    pltpu.sync_copy(bv_hbm.at[pl.ds(base, _CPW)], bvv)
    pltpu.sync_copy(ffn_hbm.at[pl.ds(base, _CPW)], fnv)
    pltpu.sync_copy(fx_hbm.at[pl.ds(base, _CPW)], fxv)

    def _row_weight(accs, bkbk, bvbv, qbk):
        aq, ab, av, ar, asm = accs
        rr = _bsum(ar)
        kk = rr + 2.0 * _bsum(ab) + bkbk
        vv = rr + 2.0 * _bsum(av) + bvbv
        s = _bsum(aq) + qbk
        sim = jnp.exp(_BETA * (s * _rsqrt(kk) - 1.0))
        return jnp.where(_bsum(asm) == 0.0, 0.0, sim * _rsqrt(vv))

    def _compute(k_idx):
        def _cc(ch, acc):
            a1, a2, a3 = acc
            sl = pl.ds(ch * 16, 16)
            qc = qv_v[0, sl]
            bkc = bkv[k_idx, sl]
            bvc = bvv[k_idx, sl]
            return (a1 + bkc * bkc, a2 + bvc * bvc, a3 + qc * bkc)
        bkbk, bvbv, qbk = lax.fori_loop(0, _NCH, _cc, (z, z, z), unroll=2)
        bkbk = _bsum(bkbk)
        bvbv = _bsum(bvbv)
        qbk = _bsum(qbk)

        def _p1_block(loads):
            nr = len(loads)

            def _p1(ch, acc):
                sl = pl.ds(ch * 16, 16)
                qc = qv_v[0, sl]
                bkc = bkv[k_idx, sl]
                bvc = bvv[k_idx, sl]
                out = []
                for j in range(nr):
                    rv = loads[j](sl)
                    aq, ab, av, ar, asm = acc[j]
                    out.append((aq + rv * qc, ab + rv * bkc, av + rv * bvc,
                                ar + rv * rv, asm + rv))
                return tuple(out)
            res = lax.fori_loop(0, _NCH, _p1, tuple((z, z, z, z, z)
                                                    for _ in range(nr)),
                                unroll=2)
            return [_row_weight(res[j], bkbk, bvbv, qbk) for j in range(nr)]

        lloads = [lambda sl: rows_v[48, sl], lambda sl: rows_v[49, sl],
                  lambda sl: fxv[k_idx, 0, sl]]
        lw = _p1_block(lloads)
        wsum = lw[0] + lw[1] + lw[2]

        def _p2l(ch, carry):
            sl = pl.ds(ch * 16, 16)
            adap_v[sl] = (rows_v[48, sl] * lw[0] +
                          rows_v[49, sl] * lw[1] + fxv[k_idx, 0, sl] * lw[2])
            return carry
        lax.fori_loop(0, _NCH, _p2l, 0, unroll=2)

        for r0 in range(0, 48, 6):
            ws = _p1_block([(lambda sl, r=r0 + j: rows_v[r, sl])
                            for j in range(6)])
            for w in ws:
                wsum = wsum + w

            def _p2(ch, carry, r0=r0, ws=ws):
                sl = pl.ds(ch * 16, 16)
                acc = adap_v[sl]
                for j in range(6):
                    acc = acc + rows_v[r0 + j, sl] * ws[j]
                adap_v[sl] = acc
                return carry
            lax.fori_loop(0, _NCH, _p2, 0, unroll=2)

        def _p3a(ch, acc):
            sl = pl.ds(ch * 16, 16)
            x = adap_v[sl] + wsum * bvv[k_idx, sl]
            return acc + x * x
        aa = _bsum(lax.fori_loop(0, _NCH, _p3a, z, unroll=2))
        r1 = _rsqrt(aa)

        def _p3b(ch, acc):
            a2, ai = acc
            sl = pl.ds(ch * 16, 16)
            x = (adap_v[sl] + wsum * bvv[k_idx, sl]) * r1 + fnv[k_idx, sl]
            return (a2 + x * x, ai + x * iv_v[0, sl])
        aa2, ai = lax.fori_loop(0, _NCH, _p3b, (z, z), unroll=2)
        lg_v[k_idx, :] = _LOGIT_SCALE * _bsum(ai) * _rsqrt(_bsum(aa2))

    def _class_body(k_idx, carry):
        pltpu.async_copy(mem_hbm.at[base + k_idx], rows_v, sem_a).wait()
        _compute(k_idx)
        return carry
    lax.fori_loop(0, _CPW, _class_body, 0)
    pltpu.sync_copy(lg_v, out_hbm.at[pl.ds(base, _CPW)])


def _tc_body(q_ref, img_ref, mem_ref, fx_ref, bk_ref, bv_ref, fn_ref, o_ref):
    mem = mem_ref[...]
    fxb = fx_ref[...]
    bk = bk_ref[...]
    bv = bv_ref[...]
    fn = fn_ref[...]

    def _b3(x, shape, dims):
        return lax.broadcast_in_dim(x, shape, dims)

    q = q_ref[...]
    bkbk = _b3(jnp.sum(bk * bk, -1), (_M, _TCB), (1,))
    bvbv = _b3(jnp.sum(bv * bv, -1), (_M, _TCB), (1,))
    qbk = _b3(jnp.sum(bk * _b3(q, (_TCB, _D), (0, 1)), -1), (_M, _TCB), (1,))

    def _w(r, n):
        rr = jnp.sum(r * r, -1)
        kk = rr + 2.0 * jnp.sum(r * _b3(bk, (n, _TCB, _D), (1, 2)), -1) + \
            bkbk[:n]
        vv = rr + 2.0 * jnp.sum(r * _b3(bv, (n, _TCB, _D), (1, 2)), -1) + \
            bvbv[:n]
        s = jnp.sum(r * _b3(q, (n, _TCB, _D), (0, 2)), -1) + qbk[:n]
        sim = jnp.exp(_BETA * (s * lax.rsqrt(kk) - 1.0))
        empty = jnp.sum(r, -1) == 0.0
        return jnp.where(empty, 0.0, sim * lax.rsqrt(vv))

    w = _w(mem, _M)
    w_f = _w(fxb, 1)
    wsum = jnp.sum(w, 0) + jnp.sum(w_f, 0)
    a = jnp.sum(mem * _b3(w, (_M, _TCB, _D), (0, 1)), 0) + \
        jnp.sum(fxb * _b3(w_f, (1, _TCB, _D), (0, 1)), 0) + \
        _b3(wsum, (_TCB, _D), (0,)) * bv
    a = a * lax.rsqrt(jnp.sum(a * a, -1, keepdims=True))
    a2 = a + fn
    img2 = _b3(img_ref[...], (_TCB, _D), (0, 1))
    o_ref[...] = _LOGIT_SCALE * jnp.sum(a2 * img2, -1, keepdims=True) * \
        lax.rsqrt(jnp.sum(a2 * a2, -1, keepdims=True))


def _q_body(img_ref, gb_ref, o_ref):
    s = jnp.sum(gb_ref[...], axis=0, keepdims=True) * (1.0 / _C) + img_ref[...]
    o_ref[...] = s * lax.rsqrt(jnp.sum(s * s))


def _softmax_body(x_ref, o_ref):
    x = x_ref[...]
    idx = lax.broadcasted_iota(jnp.int32, (8, 128), 0) * 128 + \
        lax.broadcasted_iota(jnp.int32, (8, 128), 1)
    x = jnp.where(idx < _C, x, -jnp.inf)
    e = jnp.where(idx < _C, jnp.exp(x - jnp.max(x)), 0.0)
    o_ref[...] = e / jnp.sum(e)


@jax.jit
def kernel(img_feat, image_feature_memory, fixed_global_feat_vanilla,
           global_bias, global_bias_key, global_bias_value, global_ffn_bias):
    q = pl.pallas_call(
        _q_body,
        out_shape=jax.ShapeDtypeStruct((1, _D), jnp.float32),
    )(img_feat, global_bias)

    sc = pl.kernel(
        _sc_body,
        mesh=plsc.VectorSubcoreMesh(core_axis_name="c", subcore_axis_name="s"),
        out_type=jax.ShapeDtypeStruct((_S, 16), jnp.float32),
        compiler_params=pltpu.CompilerParams(use_tc_tiling_on_sc=True,
                                             needs_layout_passes=False),
        scratch_types=[
            pltpu.VMEM((_M, _D), jnp.float32),
            pltpu.VMEM((_CPW, _D), jnp.float32),
            pltpu.VMEM((_CPW, _D), jnp.float32),
            pltpu.VMEM((_CPW, _D), jnp.float32),
            pltpu.VMEM((_CPW, 1, _D), jnp.float32),
            pltpu.VMEM((_D,), jnp.float32),
            pltpu.VMEM((1, _D), jnp.float32),
            pltpu.VMEM((1, _D), jnp.float32),
            pltpu.VMEM((_CPW, 16), jnp.float32),
            pltpu.SemaphoreType.DMA,
        ],
    )
    sc_lg16 = sc(q, image_feature_memory[:_S],
                 fixed_global_feat_vanilla[:_S], global_bias_key[:_S],
                 global_bias_value[:_S], global_ffn_bias[:_S], img_feat)

    ntc = (_C - _S) // _TCB
    tc_lg = pl.pallas_call(
        _tc_body,
        grid=(ntc,),
        in_specs=[
            pl.BlockSpec((1, _D), lambda i: (0, 0)),
            pl.BlockSpec((1, _D), lambda i: (0, 0)),
            pl.BlockSpec((_M, _TCB, _D), lambda i: (0, _S // _TCB + i, 0)),
            pl.BlockSpec((1, _TCB, _D), lambda i: (0, _S // _TCB + i, 0)),
            pl.BlockSpec((_TCB, _D), lambda i: (_S // _TCB + i, 0)),
            pl.BlockSpec((_TCB, _D), lambda i: (_S // _TCB + i, 0)),
            pl.BlockSpec((_TCB, _D), lambda i: (_S // _TCB + i, 0)),
        ],
        out_specs=pl.BlockSpec((_TCB, 1), lambda i: (i, 0)),
        out_shape=jax.ShapeDtypeStruct((_C - _S, 1), jnp.float32),
    )(q, img_feat, jnp.transpose(image_feature_memory, (1, 0, 2)),
      jnp.transpose(fixed_global_feat_vanilla, (1, 0, 2)),
      global_bias_key, global_bias_value, global_ffn_bias)

    lg = jnp.concatenate([sc_lg16[:, 0], tc_lg[:, 0],
                          jnp.zeros(1024 - _C, jnp.float32)])
    probs = pl.pallas_call(
        _softmax_body,
        out_shape=jax.ShapeDtypeStruct((8, 128), jnp.float32),
    )(lg.reshape(8, 128))
    return probs.reshape(1024)[:_C][None, :]


# speedup vs baseline: 2.4816x; 1.7336x over previous
import functools

import jax
import jax.numpy as jnp
from jax import lax
from jax.experimental import pallas as pl
from jax.experimental.pallas import tpu as pltpu
from jax.experimental.pallas import tpu_sc as plsc

_BETA = 5.5
_LOGIT_SCALE = 100.0
_C, _M, _D = 1000, 50, 1024
_NW = 32
_CPW = 8
_S = _NW * _CPW
_NCH = _D // 16
_TCB = 8


def _bsum(v):
    return jnp.broadcast_to(jnp.sum(v), (16,))


def _rsqrt(x):
    i = lax.bitcast_convert_type(x, jnp.int32)
    y = lax.bitcast_convert_type(jnp.int32(0x5F3759DF) - (i >> 1), jnp.float32)
    for _ in range(3):
        y = y * (1.5 - 0.5 * x * y * y)
    return y


_RN = 5


def _sc_body(q_hbm, mem_hbm, fx_hbm, bk_hbm, bv_hbm, ffn_hbm, img_hbm,
             out_hbm, rowsb, bkv, bvv, fnv, fxv, adap_v, ckk, cvv, cqb,
             wsum_v, qv_v, iv_v, lg_v, sem_a, sem_b):
    wid = lax.axis_index("s") * 2 + lax.axis_index("c")
    z = jnp.zeros((16,), jnp.float32)
    base = wid * _CPW

    pltpu.sync_copy(q_hbm, qv_v)
    pltpu.sync_copy(img_hbm, iv_v)
    pltpu.sync_copy(bk_hbm.at[pl.ds(base, _CPW)], bkv)
    pltpu.sync_copy(bv_hbm.at[pl.ds(base, _CPW)], bvv)
    pltpu.sync_copy(ffn_hbm.at[pl.ds(base, _CPW)], fnv)
    pltpu.sync_copy(fx_hbm.at[:, pl.ds(base, _CPW)], fxv)
    pltpu.async_copy(mem_hbm.at[pl.ds(0, _RN), pl.ds(base, _CPW)],
                     rowsb.at[0], sem_a)

    def _bdot(v):
        return jnp.broadcast_to(jnp.sum(v), (16,))

    def _const_cls(c, carry):
        def _cc(ch, acc):
            a1, a2, a3 = acc
            sl = pl.ds(ch * 16, 16)
            qc = qv_v[0, sl]
            bkc = bkv[c, sl]
            bvc = bvv[c, sl]
            return (a1 + bkc * bkc, a2 + bvc * bvc, a3 + qc * bkc)
        a1, a2, a3 = lax.fori_loop(0, _NCH, _cc, (z, z, z), unroll=2)
        ckk[c, :] = _bdot(a1)
        cvv[c, :] = _bdot(a2)
        cqb[c, :] = _bdot(a3)
        wsum_v[c, :] = z

        def _za(ch, carry2):
            adap_v[c, pl.ds(ch * 16, 16)] = z
            return carry2
        return lax.fori_loop(0, _NCH, _za, carry)
    lax.fori_loop(0, _CPW, _const_cls, 0)

    def _row_weight(accs, c):
        aq, ab, av, ar, asm = accs
        rr = _bdot(ar)
        kk = rr + 2.0 * _bdot(ab) + ckk[c, :]
        vv = rr + 2.0 * _bdot(av) + cvv[c, :]
        s = _bdot(aq) + cqb[c, :]
        sim = jnp.exp(_BETA * (s * _rsqrt(kk) - 1.0))
        return jnp.where(_bdot(asm) == 0.0, 0.0, sim * _rsqrt(vv))

    def _chunk(b):
        def _cls(c, carry):
            def _p1(ch, acc):
                sl = pl.ds(ch * 16, 16)
                qc = qv_v[0, sl]
                bkc = bkv[c, sl]
                bvc = bvv[c, sl]
                out = []
                for r in range(_RN):
                    rv = rowsb[b, r, c, sl]
                    aq, ab, av, ar, asm = acc[r]
                    out.append((aq + rv * qc, ab + rv * bkc, av + rv * bvc,
                                ar + rv * rv, asm + rv))
                return tuple(out)
            res = lax.fori_loop(0, _NCH, _p1, tuple((z, z, z, z, z)
                                                    for _ in range(_RN)),
                                unroll=2)
            ws = [_row_weight(res[r], c) for r in range(_RN)]
            wacc = wsum_v[c, :]
            for w in ws:
                wacc = wacc + w
            wsum_v[c, :] = wacc

            def _p2(ch, carry2):
                sl = pl.ds(ch * 16, 16)
                acc = adap_v[c, sl]
                for r in range(_RN):
                    acc = acc + rowsb[b, r, c, sl] * ws[r]
                adap_v[c, sl] = acc
                return carry2
            return lax.fori_loop(0, _NCH, _p2, carry)
        lax.fori_loop(0, _CPW, _cls, 0)

    def _start(k, sem):
        pltpu.async_copy(
            mem_hbm.at[pl.ds(k * _RN, _RN), pl.ds(base, _CPW)],
            rowsb.at[k % 2], sem)

    def _wait(k, sem):
        pltpu.make_async_copy(
            mem_hbm.at[pl.ds(k * _RN, _RN), pl.ds(base, _CPW)],
            rowsb.at[k % 2], sem).wait()

    def _pair(i, carry):
        k0 = 2 * i

        @pl.when(k0 + 1 < _M // _RN)
        def _():
            _start(k0 + 1, sem_b)
        _wait(k0, sem_a)
        _chunk(0)

        @pl.when(k0 + 2 < _M // _RN)
        def _():
            _start(k0 + 2, sem_a)

        @pl.when(k0 + 1 < _M // _RN)
        def _():
            _wait(k0 + 1, sem_b)
            _chunk(1)
        return carry
    lax.fori_loop(0, (_M // _RN + 1) // 2, _pair, 0)

    def _fin(c, carry):
        def _pf(ch, acc):
            sl = pl.ds(ch * 16, 16)
            qc = qv_v[0, sl]
            bkc = bkv[c, sl]
            bvc = bvv[c, sl]
            rv = fxv[0, c, sl]
            aq, ab, av, ar, asm = acc
            return (aq + rv * qc, ab + rv * bkc, av + rv * bvc,
                    ar + rv * rv, asm + rv)
        res = lax.fori_loop(0, _NCH, _pf, (z, z, z, z, z), unroll=2)
        wf = _row_weight(res, c)
        wsum = wsum_v[c, :] + wf

        def _p3a(ch, acc):
            sl = pl.ds(ch * 16, 16)
            x = adap_v[c, sl] + wf * fxv[0, c, sl] + wsum * bvv[c, sl]
            return acc + x * x
        aa = _bdot(lax.fori_loop(0, _NCH, _p3a, z, unroll=2))
        r1 = _rsqrt(aa)

        def _p3b(ch, acc):
            a2, ai = acc
            sl = pl.ds(ch * 16, 16)
            x = (adap_v[c, sl] + wf * fxv[0, c, sl] +
                 wsum * bvv[c, sl]) * r1 + fnv[c, sl]
            return (a2 + x * x, ai + x * iv_v[0, sl])
        aa2, ai = lax.fori_loop(0, _NCH, _p3b, (z, z), unroll=2)
        lg_v[c, :] = _LOGIT_SCALE * _bdot(ai) * _rsqrt(_bdot(aa2))
        return carry
    lax.fori_loop(0, _CPW, _fin, 0)

    pltpu.sync_copy(lg_v, out_hbm.at[pl.ds(base, _CPW)])


def _tc_body(q_ref, img_ref, mem_ref, fx_ref, bk_ref, bv_ref, fn_ref, o_ref):
    mem = mem_ref[...]
    fxb = fx_ref[...]
    bk = bk_ref[...]
    bv = bv_ref[...]
    fn = fn_ref[...]

    def _b3(x, shape, dims):
        return lax.broadcast_in_dim(x, shape, dims)

    q = q_ref[...]
    bkbk = _b3(jnp.sum(bk * bk, -1), (_M, _TCB), (1,))
    bvbv = _b3(jnp.sum(bv * bv, -1), (_M, _TCB), (1,))
    qbk = _b3(jnp.sum(bk * _b3(q, (_TCB, _D), (0, 1)), -1), (_M, _TCB), (1,))

    def _w(r, n):
        rr = jnp.sum(r * r, -1)
        kk = rr + 2.0 * jnp.sum(r * _b3(bk, (n, _TCB, _D), (1, 2)), -1) + \
            bkbk[:n]
        vv = rr + 2.0 * jnp.sum(r * _b3(bv, (n, _TCB, _D), (1, 2)), -1) + \
            bvbv[:n]
        s = jnp.sum(r * _b3(q, (n, _TCB, _D), (0, 2)), -1) + qbk[:n]
        sim = jnp.exp(_BETA * (s * lax.rsqrt(kk) - 1.0))
        empty = jnp.sum(r, -1) == 0.0
        return jnp.where(empty, 0.0, sim * lax.rsqrt(vv))

    w = _w(mem, _M)
    w_f = _w(fxb, 1)
    wsum = jnp.sum(w, 0) + jnp.sum(w_f, 0)
    a = jnp.sum(mem * _b3(w, (_M, _TCB, _D), (0, 1)), 0) + \
        jnp.sum(fxb * _b3(w_f, (1, _TCB, _D), (0, 1)), 0) + \
        _b3(wsum, (_TCB, _D), (0,)) * bv
    a = a * lax.rsqrt(jnp.sum(a * a, -1, keepdims=True))
    a2 = a + fn
    img2 = _b3(img_ref[...], (_TCB, _D), (0, 1))
    o_ref[...] = _LOGIT_SCALE * jnp.sum(a2 * img2, -1, keepdims=True) * \
        lax.rsqrt(jnp.sum(a2 * a2, -1, keepdims=True))


def _q_body(img_ref, gb_ref, o_ref):
    s = jnp.sum(gb_ref[...], axis=0, keepdims=True) * (1.0 / _C) + img_ref[...]
    o_ref[...] = s * lax.rsqrt(jnp.sum(s * s))


def _softmax_body(x_ref, o_ref):
    x = x_ref[...]
    idx = lax.broadcasted_iota(jnp.int32, (8, 128), 0) * 128 + \
        lax.broadcasted_iota(jnp.int32, (8, 128), 1)
    x = jnp.where(idx < _C, x, -jnp.inf)
    e = jnp.where(idx < _C, jnp.exp(x - jnp.max(x)), 0.0)
    o_ref[...] = e / jnp.sum(e)


@jax.jit
def kernel(img_feat, image_feature_memory, fixed_global_feat_vanilla,
           global_bias, global_bias_key, global_bias_value, global_ffn_bias):
    q = pl.pallas_call(
        _q_body,
        out_shape=jax.ShapeDtypeStruct((1, _D), jnp.float32),
    )(img_feat, global_bias)

    sc = pl.kernel(
        _sc_body,
        mesh=plsc.VectorSubcoreMesh(core_axis_name="c", subcore_axis_name="s"),
        out_type=jax.ShapeDtypeStruct((_S, 16), jnp.float32),
        compiler_params=pltpu.CompilerParams(use_tc_tiling_on_sc=True,
                                             needs_layout_passes=False),
        scratch_types=[
            pltpu.VMEM((2, _RN, _CPW, _D), jnp.float32),
            pltpu.VMEM((_CPW, _D), jnp.float32),
            pltpu.VMEM((_CPW, _D), jnp.float32),
            pltpu.VMEM((_CPW, _D), jnp.float32),
            pltpu.VMEM((1, _CPW, _D), jnp.float32),
            pltpu.VMEM((_CPW, _D), jnp.float32),
            pltpu.VMEM((_CPW, 16), jnp.float32),
            pltpu.VMEM((_CPW, 16), jnp.float32),
            pltpu.VMEM((_CPW, 16), jnp.float32),
            pltpu.VMEM((_CPW, 16), jnp.float32),
            pltpu.VMEM((1, _D), jnp.float32),
            pltpu.VMEM((1, _D), jnp.float32),
            pltpu.VMEM((_CPW, 16), jnp.float32),
            pltpu.SemaphoreType.DMA,
            pltpu.SemaphoreType.DMA,
        ],
    )
    sc_lg16 = sc(q, jnp.transpose(image_feature_memory, (1, 0, 2)),
                 jnp.transpose(fixed_global_feat_vanilla, (1, 0, 2)),
                 global_bias_key, global_bias_value, global_ffn_bias,
                 img_feat)

    ntc = (_C - _S) // _TCB
    tc_lg = pl.pallas_call(
        _tc_body,
        grid=(ntc,),
        in_specs=[
            pl.BlockSpec((1, _D), lambda i: (0, 0)),
            pl.BlockSpec((1, _D), lambda i: (0, 0)),
            pl.BlockSpec((_M, _TCB, _D), lambda i: (0, _S // _TCB + i, 0)),
            pl.BlockSpec((1, _TCB, _D), lambda i: (0, _S // _TCB + i, 0)),
            pl.BlockSpec((_TCB, _D), lambda i: (_S // _TCB + i, 0)),
            pl.BlockSpec((_TCB, _D), lambda i: (_S // _TCB + i, 0)),
            pl.BlockSpec((_TCB, _D), lambda i: (_S // _TCB + i, 0)),
        ],
        out_specs=pl.BlockSpec((_TCB, 1), lambda i: (i, 0)),
        out_shape=jax.ShapeDtypeStruct((_C - _S, 1), jnp.float32),
    )(q, img_feat, jnp.transpose(image_feature_memory, (1, 0, 2)),
      jnp.transpose(fixed_global_feat_vanilla, (1, 0, 2)),
      global_bias_key, global_bias_value, global_ffn_bias)

    lg = jnp.concatenate([sc_lg16[:, 0], tc_lg[:, 0],
                          jnp.zeros(1024 - _C, jnp.float32)])
    probs = pl.pallas_call(
        _softmax_body,
        out_shape=jax.ShapeDtypeStruct((8, 128), jnp.float32),
    )(lg.reshape(8, 128))
    return probs.reshape(1024)[:_C][None, :]
